# Initial kernel scaffold; baseline (speedup 1.0000x reference)
import jax
import jax.numpy as jnp
from jax.experimental import pallas as pl


def kernel(x, params, edge_index, batch):
    raise NotImplementedError("write your pallas kernel here")


# speedup vs baseline: 3.7543x; 3.7543x over previous
import functools

import jax
import jax.numpy as jnp
import numpy as np
from jax import lax
from jax.experimental import pallas as pl
from jax.experimental.pallas import tpu as pltpu
from jax.experimental.pallas import tpu_sc as plsc

N = 10000
E = 320000
D = 128
L = 3
G = 64
LAT = 64

BLK = 1000
NB = N // BLK

NC = 2
NS = 16
EPW = E // (NC * NS)
EK = 80
ZR = 624
ZTAIL = N - NS * ZR

_INV_BN = 1.0 / np.sqrt(1.0 + 1e-5)


def _gelu(x):
    return x * 0.5 * (1.0 + lax.erf(x * np.float32(1.0 / np.sqrt(2.0))))


def _ln(x, g, b):
    m = jnp.mean(x, axis=-1, keepdims=True)
    v = jnp.mean((x - m) ** 2, axis=-1, keepdims=True)
    return (x - m) / jnp.sqrt(v + 1e-5) * g + b


def _members(batch_ref):
    bv = batch_ref[0]
    gi = lax.broadcasted_iota(jnp.int32, (G, BLK), 0)
    return (gi == bv).astype(jnp.float32)


def _pre_body(x_ref, g_ref, b_ref, batch_ref, x0_ref, pooled_ref):
    i = pl.program_id(0)
    x0 = x_ref[...] * (g_ref[...] * _INV_BN) + b_ref[...]
    x0_ref[...] = x0
    ohT = _members(batch_ref)
    contrib = jnp.dot(ohT, x0, preferred_element_type=jnp.float32)

    @pl.when(i == 0)
    def _():
        pooled_ref[...] = contrib

    @pl.when(i > 0)
    def _():
        pooled_ref[...] += contrib


def _pre(x, g, b, batch3):
    return pl.pallas_call(
        _pre_body,
        grid=(NB,),
        in_specs=[
            pl.BlockSpec((BLK, D), lambda i: (i, 0)),
            pl.BlockSpec((1, D), lambda i: (0, 0)),
            pl.BlockSpec((1, D), lambda i: (0, 0)),
            pl.BlockSpec((1, 1, BLK), lambda i: (i, 0, 0)),
        ],
        out_specs=[
            pl.BlockSpec((BLK, D), lambda i: (i, 0)),
            pl.BlockSpec((G, D), lambda i: (0, 0)),
        ],
        out_shape=[
            jax.ShapeDtypeStruct((N, D), jnp.float32),
            jax.ShapeDtypeStruct((G, D), jnp.float32),
        ],
    )(x, g, b, batch3)


def _vn_body(pooled_ref, w1_ref, b1_ref, w2_ref, b2_ref, vnu_ref):
    h = jnp.dot(pooled_ref[...], w1_ref[...],
                preferred_element_type=jnp.float32) + b1_ref[...]
    h = _gelu(h)
    vnu_ref[...] = jnp.dot(h, w2_ref[...],
                           preferred_element_type=jnp.float32) + b2_ref[...]


def _vn_mlp(pooled, w1, b1, w2, b2):
    return pl.pallas_call(
        _vn_body,
        out_shape=jax.ShapeDtypeStruct((G, D), jnp.float32),
    )(pooled, w1, b1, w2, b2)


def _vnadd_body(x_ref, vnu_ref, batch_ref, out_ref):
    ohT = _members(batch_ref)
    upd = lax.dot_general(ohT, vnu_ref[...], (((0,), (0,)), ((), ())),
                          preferred_element_type=jnp.float32)
    out_ref[...] = x_ref[...] + upd


def _vn_add(x, vnu, batch3):
    return pl.pallas_call(
        _vnadd_body,
        grid=(NB,),
        in_specs=[
            pl.BlockSpec((BLK, D), lambda i: (i, 0)),
            pl.BlockSpec((G, D), lambda i: (0, 0)),
            pl.BlockSpec((1, 1, BLK), lambda i: (i, 0, 0)),
        ],
        out_specs=pl.BlockSpec((BLK, D), lambda i: (i, 0)),
        out_shape=jax.ShapeDtypeStruct((N, D), jnp.float32),
    )(x, vnu, batch3)


def _dense_body(x_ref, aggr_ref, batch_ref, eps_ref,
                gw_ref, gb_ref, w1_ref, b1_ref, n1g_ref, n1b_ref,
                w2_ref, b2_ref, n2g_ref, n2b_ref, bng_ref, bnb_ref,
                jkw_ref, jkb_ref,
                xn_ref, proj_ref, pooled_ref):
    i = pl.program_id(0)
    h = x_ref[...] * eps_ref[...] + aggr_ref[0] + aggr_ref[1]
    gate = jax.nn.sigmoid(
        jnp.dot(h, gw_ref[...], preferred_element_type=jnp.float32)
        + gb_ref[...])
    t = jnp.dot(h, w1_ref[...], preferred_element_type=jnp.float32) + b1_ref[...]
    t = _gelu(_ln(t, n1g_ref[...], n1b_ref[...]))
    t = jnp.dot(t, w2_ref[...], preferred_element_type=jnp.float32) + b2_ref[...]
    t = _gelu(_ln(t, n2g_ref[...], n2b_ref[...]))
    xn = gate * t + (1.0 - gate) * h
    xn = xn * (bng_ref[...] * _INV_BN) + bnb_ref[...]
    xn = _gelu(xn)
    xn_ref[...] = xn
    proj_ref[...] = jnp.dot(xn, jkw_ref[...],
                            preferred_element_type=jnp.float32) + jkb_ref[...]
    ohT = _members(batch_ref)
    contrib = jnp.dot(ohT, xn, preferred_element_type=jnp.float32)

    @pl.when(i == 0)
    def _():
        pooled_ref[...] = contrib

    @pl.when(i > 0)
    def _():
        pooled_ref[...] += contrib


def _dense(x, aggr, batch3, eps_row, gw, gb, w1, b1, n1g, n1b,
           w2, b2, n2g, n2b, bng, bnb, jkw, jkb):
    vec = pl.BlockSpec((1, D), lambda i: (0, 0))
    mat = pl.BlockSpec((D, D), lambda i: (0, 0))
    return pl.pallas_call(
        _dense_body,
        grid=(NB,),
        in_specs=[
            pl.BlockSpec((BLK, D), lambda i: (i, 0)),
            pl.BlockSpec((2, BLK, D), lambda i: (0, i, 0)),
            pl.BlockSpec((1, 1, BLK), lambda i: (i, 0, 0)),
            vec, mat, vec, mat, vec, vec, vec, mat, vec, vec, vec, vec, vec,
            mat, vec,
        ],
        out_specs=[
            pl.BlockSpec((BLK, D), lambda i: (i, 0)),
            pl.BlockSpec((BLK, D), lambda i: (i, 0)),
            pl.BlockSpec((G, D), lambda i: (0, 0)),
        ],
        out_shape=[
            jax.ShapeDtypeStruct((N, D), jnp.float32),
            jax.ShapeDtypeStruct((N, D), jnp.float32),
            jax.ShapeDtypeStruct((G, D), jnp.float32),
        ],
    )(x, aggr, batch3, eps_row, gw, gb, w1, b1, n1g, n1b, w2, b2, n2g, n2b,
      bng, bnb, jkw, jkb)


def _jkpool_body(p0_ref, p1_ref, p2_ref, aw_ref, batch_ref,
                 add_ref, cnt_ref, mx_ref):
    i = pl.program_id(0)
    a0 = p0_ref[...]
    a1 = p1_ref[...]
    a2 = p2_ref[...]
    aw = aw_ref[...]
    l0 = jnp.sum(a0 * aw[0:1, :], axis=-1, keepdims=True)
    l1 = jnp.sum(a1 * aw[1:2, :], axis=-1, keepdims=True)
    l2 = jnp.sum(a2 * aw[2:3, :], axis=-1, keepdims=True)
    m = jnp.maximum(jnp.maximum(l0, l1), l2)
    e0 = jnp.exp(l0 - m)
    e1 = jnp.exp(l1 - m)
    e2 = jnp.exp(l2 - m)
    s = e0 + e1 + e2
    xf = (e0 * a0 + e1 * a1 + e2 * a2) / s
    ohT = _members(batch_ref)
    addc = jnp.dot(ohT, xf, preferred_element_type=jnp.float32)
    cntc = jnp.dot(ohT, jnp.ones((BLK, D), jnp.float32),
                   preferred_element_type=jnp.float32)

    @pl.when(i == 0)
    def _():
        add_ref[...] = addc
        cnt_ref[...] = cntc
        mx_ref[...] = jnp.full((G, D), -jnp.inf, jnp.float32)

    @pl.when(i > 0)
    def _():
        add_ref[...] += addc
        cnt_ref[...] += cntc

    def body(g, carry):
        eg = (lax.broadcasted_iota(jnp.int32, (G, 1), 0) == g
              ).astype(jnp.float32)
        maskcol = lax.dot_general(ohT, eg, (((0,), (0,)), ((), ())),
                                  preferred_element_type=jnp.float32)
        col = jnp.where(maskcol > 0.5, xf, -jnp.inf)
        gm = jnp.max(col, axis=0, keepdims=True)
        mx_ref[pl.ds(g, 1), :] = jnp.maximum(mx_ref[pl.ds(g, 1), :], gm)
        return carry

    lax.fori_loop(0, G, body, 0)


def _jk_pool(p0, p1, p2, awp, batch3):
    return pl.pallas_call(
        _jkpool_body,
        grid=(NB,),
        in_specs=[
            pl.BlockSpec((BLK, D), lambda i: (i, 0)),
            pl.BlockSpec((BLK, D), lambda i: (i, 0)),
            pl.BlockSpec((BLK, D), lambda i: (i, 0)),
            pl.BlockSpec((8, D), lambda i: (0, 0)),
            pl.BlockSpec((1, 1, BLK), lambda i: (i, 0, 0)),
        ],
        out_specs=[
            pl.BlockSpec((G, D), lambda i: (0, 0)),
            pl.BlockSpec((G, D), lambda i: (0, 0)),
            pl.BlockSpec((G, D), lambda i: (0, 0)),
        ],
        out_shape=[
            jax.ShapeDtypeStruct((G, D), jnp.float32),
            jax.ShapeDtypeStruct((G, D), jnp.float32),
            jax.ShapeDtypeStruct((G, D), jnp.float32),
        ],
    )(p0, p1, p2, awp, batch3)


def _final_body(add_ref, cnt_ref, mx_ref, pw_ref, fc1w_ref, fc1b_ref,
                lng_ref, lnb_ref, fc2w_ref, fc2b_ref, out_ref):
    pw = pw_ref[...]
    m = jnp.max(pw, axis=-1, keepdims=True)
    e = jnp.exp(pw - m)
    sm = e / jnp.sum(e, axis=-1, keepdims=True)
    add = add_ref[...]
    mean = add / jnp.maximum(cnt_ref[...], 1.0)
    pooled = (add * sm[0:1, 0:1] + mean * sm[0:1, 1:2]
              + mx_ref[...] * sm[0:1, 2:3])
    o = jnp.dot(pooled, fc1w_ref[...],
                preferred_element_type=jnp.float32) + fc1b_ref[...]
    o = _gelu(_ln(o, lng_ref[...], lnb_ref[...]))
    o = o + pooled
    out_ref[...] = jnp.dot(o, fc2w_ref[...],
                           preferred_element_type=jnp.float32) + fc2b_ref[...]


def _final(add, cnt, mx, pwp, fc1w, fc1b, lng, lnb, fc2w, fc2b):
    return pl.pallas_call(
        _final_body,
        out_shape=jax.ShapeDtypeStruct((G, LAT), jnp.float32),
    )(add, cnt, mx, pwp, fc1w, fc1b, lng, lnb, fc2w, fc2b)


def _sc_aggregate(x, src, dst, zeros):
    mesh = plsc.VectorSubcoreMesh(core_axis_name="c", subcore_axis_name="s")

    @functools.partial(
        pl.kernel,
        out_type=jax.ShapeDtypeStruct((NC, N, D), jnp.float32),
        mesh=mesh,
        scratch_types=[
            pltpu.VMEM((EK,), jnp.int32),
            pltpu.VMEM((EK,), jnp.int32),
            pltpu.VMEM((EK, D), jnp.float32),
            pltpu.VMEM_SHARED((N, D), jnp.float32),
            pltpu.SemaphoreType.DMA,
        ],
    )
    def agg(x_hbm, src_hbm, dst_hbm, zero_hbm, out_hbm,
            src_v, dst_v, rows_v, acc_sh, sem):
        c = lax.axis_index("c")
        s = lax.axis_index("s")
        w = c * NS + s
        pltpu.sync_copy(zero_hbm, acc_sh.at[pl.ds(s * ZR, ZR)])

        @pl.when(s == 0)
        def _():
            pltpu.sync_copy(zero_hbm.at[pl.ds(0, ZTAIL)],
                            acc_sh.at[pl.ds(NS * ZR, ZTAIL)])

        plsc.subcore_barrier()
        ebase = w * EPW

        def body(i, carry):
            base = ebase + i * EK
            pltpu.sync_copy(src_hbm.at[pl.ds(base, EK)], src_v)
            pltpu.sync_copy(dst_hbm.at[pl.ds(base, EK)], dst_v)
            pltpu.async_copy(x_hbm.at[src_v], rows_v, sem).wait()
            pltpu.sync_copy(rows_v, acc_sh.at[dst_v], add=True)
            return carry

        lax.fori_loop(0, EPW // EK, body, 0)
        plsc.subcore_barrier()
        pltpu.sync_copy(acc_sh.at[pl.ds(s * ZR, ZR)],
                        out_hbm.at[c, pl.ds(s * ZR, ZR)])

        @pl.when(s == 0)
        def _():
            pltpu.sync_copy(acc_sh.at[pl.ds(NS * ZR, ZTAIL)],
                            out_hbm.at[c, pl.ds(NS * ZR, ZTAIL)])

    return agg(x, src, dst, zeros)


def kernel(x, params, edge_index, batch):
    p = params
    src = edge_index[0]
    dst = edge_index[1]
    batch3 = batch.astype(jnp.int32).reshape(NB, 1, BLK)
    zeros = jnp.zeros((ZR, D), jnp.float32)

    def row(v):
        return v.reshape(1, -1)

    x0, pooled = _pre(x, row(p['in_bn_g']), row(p['in_bn_b']), batch3)
    projs = []
    for l in range(L):
        vnu = _vn_mlp(pooled, p[f'vn_w1_{l}'], row(p[f'vn_b1_{l}']),
                      p[f'vn_w2_{l}'], row(p[f'vn_b2_{l}']))
        xu = _vn_add(x0, vnu, batch3)
        aggr = _sc_aggregate(xu, src, dst, zeros)
        eps_row = jnp.broadcast_to((1.0 + p[f'eps{l}'])[None, None], (1, D))
        x0, proj, pooled = _dense(
            xu, aggr, batch3, eps_row,
            p[f'gate_w{l}'], row(p[f'gate_b{l}']),
            p[f'fc1_w{l}'], row(p[f'fc1_b{l}']),
            row(p[f'n1_g{l}']), row(p[f'n1_b{l}']),
            p[f'fc2_w{l}'], row(p[f'fc2_b{l}']),
            row(p[f'n2_g{l}']), row(p[f'n2_b{l}']),
            row(p[f'bn_g{l}']), row(p[f'bn_b{l}']),
            p[f'jk_w{l}'], row(p[f'jk_b{l}']))
        projs.append(proj)

    awp = jnp.zeros((8, D), jnp.float32).at[:L].set(p['attn_w'])
    pwp = jnp.full((1, D), -jnp.inf, jnp.float32).at[0, :3].set(p['pool_w'])
    add, cnt, mx = _jk_pool(projs[0], projs[1], projs[2], awp, batch3)
    return _final(add, cnt, mx, pwp, p['fc1_w'], row(p['fc1_b']),
                  row(p['ln_g']), row(p['ln_b']), p['fc2_w'],
                  row(p['fc2_b']))


# speedup vs baseline: 6.5408x; 1.7422x over previous
import functools

import jax
import jax.numpy as jnp
import numpy as np
from jax import lax
from jax.experimental import pallas as pl
from jax.experimental.pallas import tpu as pltpu
from jax.experimental.pallas import tpu_sc as plsc

N = 10000
E = 320000
D = 128
L = 3
G = 64
LAT = 64

BLK = 1000
NB = N // BLK

NC = 2
NS = 16
NW = NC * NS
EPW = E // NW
EK = 100
ECH = EPW // EK
ZR = 624
ZTAIL = N - NS * ZR

_INV_BN = 1.0 / np.sqrt(1.0 + 1e-5)


def _gelu(x):
    return x * 0.5 * (1.0 + lax.erf(x * np.float32(1.0 / np.sqrt(2.0))))


def _ln(x, g, b):
    m = jnp.mean(x, axis=-1, keepdims=True)
    v = jnp.mean((x - m) ** 2, axis=-1, keepdims=True)
    return (x - m) / jnp.sqrt(v + 1e-5) * g + b


def _members(batch_ref):
    bv = batch_ref[0]
    gi = lax.broadcasted_iota(jnp.int32, (G, BLK), 0)
    return (gi == bv).astype(jnp.float32)


def _pre_body(x_ref, g_ref, b_ref, batch_ref, x0_ref, pooled_ref):
    i = pl.program_id(0)
    x0 = x_ref[...] * (g_ref[...] * _INV_BN) + b_ref[...]
    x0_ref[...] = x0
    ohT = _members(batch_ref)
    contrib = jnp.dot(ohT, x0, preferred_element_type=jnp.float32)

    @pl.when(i == 0)
    def _():
        pooled_ref[...] = contrib

    @pl.when(i > 0)
    def _():
        pooled_ref[...] += contrib


def _pre(x, g, b, batch3):
    return pl.pallas_call(
        _pre_body,
        grid=(NB,),
        in_specs=[
            pl.BlockSpec((BLK, D), lambda i: (i, 0)),
            pl.BlockSpec((1, D), lambda i: (0, 0)),
            pl.BlockSpec((1, D), lambda i: (0, 0)),
            pl.BlockSpec((1, 1, BLK), lambda i: (i, 0, 0)),
        ],
        out_specs=[
            pl.BlockSpec((BLK, D), lambda i: (i, 0)),
            pl.BlockSpec((G, D), lambda i: (0, 0)),
        ],
        out_shape=[
            jax.ShapeDtypeStruct((N, D), jnp.float32),
            jax.ShapeDtypeStruct((G, D), jnp.float32),
        ],
    )(x, g, b, batch3)


def _vn_body(pooled_ref, w1_ref, b1_ref, w2_ref, b2_ref, vnu_ref):
    h = jnp.dot(pooled_ref[...], w1_ref[...],
                preferred_element_type=jnp.float32) + b1_ref[...]
    h = _gelu(h)
    vnu_ref[...] = jnp.dot(h, w2_ref[...],
                           preferred_element_type=jnp.float32) + b2_ref[...]


def _vn_mlp(pooled, w1, b1, w2, b2):
    return pl.pallas_call(
        _vn_body,
        out_shape=jax.ShapeDtypeStruct((G, D), jnp.float32),
    )(pooled, w1, b1, w2, b2)


def _vnadd_body(x_ref, vnu_ref, batch_ref, out_ref):
    ohT = _members(batch_ref)
    upd = lax.dot_general(ohT, vnu_ref[...], (((0,), (0,)), ((), ())),
                          preferred_element_type=jnp.float32)
    out_ref[...] = x_ref[...] + upd


def _vn_add(x, vnu, batch3):
    return pl.pallas_call(
        _vnadd_body,
        grid=(NB,),
        in_specs=[
            pl.BlockSpec((BLK, D), lambda i: (i, 0)),
            pl.BlockSpec((G, D), lambda i: (0, 0)),
            pl.BlockSpec((1, 1, BLK), lambda i: (i, 0, 0)),
        ],
        out_specs=pl.BlockSpec((BLK, D), lambda i: (i, 0)),
        out_shape=jax.ShapeDtypeStruct((N, D), jnp.float32),
    )(x, vnu, batch3)


def _dense_body(x_ref, aggr_ref, batch_ref, eps_ref,
                gw_ref, gb_ref, w1_ref, b1_ref, n1g_ref, n1b_ref,
                w2_ref, b2_ref, n2g_ref, n2b_ref, bng_ref, bnb_ref,
                jkw_ref, jkb_ref,
                xn_ref, proj_ref, pooled_ref):
    i = pl.program_id(0)
    h = x_ref[...] * eps_ref[...] + aggr_ref[0] + aggr_ref[1]
    gate = jax.nn.sigmoid(
        jnp.dot(h, gw_ref[...], preferred_element_type=jnp.float32)
        + gb_ref[...])
    t = jnp.dot(h, w1_ref[...], preferred_element_type=jnp.float32) + b1_ref[...]
    t = _gelu(_ln(t, n1g_ref[...], n1b_ref[...]))
    t = jnp.dot(t, w2_ref[...], preferred_element_type=jnp.float32) + b2_ref[...]
    t = _gelu(_ln(t, n2g_ref[...], n2b_ref[...]))
    xn = gate * t + (1.0 - gate) * h
    xn = xn * (bng_ref[...] * _INV_BN) + bnb_ref[...]
    xn = _gelu(xn)
    xn_ref[...] = xn
    proj_ref[...] = jnp.dot(xn, jkw_ref[...],
                            preferred_element_type=jnp.float32) + jkb_ref[...]
    ohT = _members(batch_ref)
    contrib = jnp.dot(ohT, xn, preferred_element_type=jnp.float32)

    @pl.when(i == 0)
    def _():
        pooled_ref[...] = contrib

    @pl.when(i > 0)
    def _():
        pooled_ref[...] += contrib


def _dense(x, aggr, batch3, eps_row, gw, gb, w1, b1, n1g, n1b,
           w2, b2, n2g, n2b, bng, bnb, jkw, jkb):
    vec = pl.BlockSpec((1, D), lambda i: (0, 0))
    mat = pl.BlockSpec((D, D), lambda i: (0, 0))
    return pl.pallas_call(
        _dense_body,
        grid=(NB,),
        in_specs=[
            pl.BlockSpec((BLK, D), lambda i: (i, 0)),
            pl.BlockSpec((2, BLK, D), lambda i: (0, i, 0)),
            pl.BlockSpec((1, 1, BLK), lambda i: (i, 0, 0)),
            vec, mat, vec, mat, vec, vec, vec, mat, vec, vec, vec, vec, vec,
            mat, vec,
        ],
        out_specs=[
            pl.BlockSpec((BLK, D), lambda i: (i, 0)),
            pl.BlockSpec((BLK, D), lambda i: (i, 0)),
            pl.BlockSpec((G, D), lambda i: (0, 0)),
        ],
        out_shape=[
            jax.ShapeDtypeStruct((N, D), jnp.float32),
            jax.ShapeDtypeStruct((N, D), jnp.float32),
            jax.ShapeDtypeStruct((G, D), jnp.float32),
        ],
    )(x, aggr, batch3, eps_row, gw, gb, w1, b1, n1g, n1b, w2, b2, n2g, n2b,
      bng, bnb, jkw, jkb)


def _jkpool_body(p0_ref, p1_ref, p2_ref, aw_ref, batch_ref,
                 add_ref, cnt_ref, mx_ref):
    i = pl.program_id(0)
    a0 = p0_ref[...]
    a1 = p1_ref[...]
    a2 = p2_ref[...]
    aw = aw_ref[...]
    l0 = jnp.sum(a0 * aw[0:1, :], axis=-1, keepdims=True)
    l1 = jnp.sum(a1 * aw[1:2, :], axis=-1, keepdims=True)
    l2 = jnp.sum(a2 * aw[2:3, :], axis=-1, keepdims=True)
    m = jnp.maximum(jnp.maximum(l0, l1), l2)
    e0 = jnp.exp(l0 - m)
    e1 = jnp.exp(l1 - m)
    e2 = jnp.exp(l2 - m)
    s = e0 + e1 + e2
    xf = (e0 * a0 + e1 * a1 + e2 * a2) / s
    ohT = _members(batch_ref)
    addc = jnp.dot(ohT, xf, preferred_element_type=jnp.float32)
    cntc = jnp.dot(ohT, jnp.ones((BLK, D), jnp.float32),
                   preferred_element_type=jnp.float32)

    @pl.when(i == 0)
    def _():
        add_ref[...] = addc
        cnt_ref[...] = cntc
        mx_ref[...] = jnp.full((G, D), -jnp.inf, jnp.float32)

    @pl.when(i > 0)
    def _():
        add_ref[...] += addc
        cnt_ref[...] += cntc

    def body(g, carry):
        eg = (lax.broadcasted_iota(jnp.int32, (G, 1), 0) == g
              ).astype(jnp.float32)
        maskcol = lax.dot_general(ohT, eg, (((0,), (0,)), ((), ())),
                                  preferred_element_type=jnp.float32)
        col = jnp.where(maskcol > 0.5, xf, -jnp.inf)
        gm = jnp.max(col, axis=0, keepdims=True)
        mx_ref[pl.ds(g, 1), :] = jnp.maximum(mx_ref[pl.ds(g, 1), :], gm)
        return carry

    lax.fori_loop(0, G, body, 0)


def _jk_pool(p0, p1, p2, awp, batch3):
    return pl.pallas_call(
        _jkpool_body,
        grid=(NB,),
        in_specs=[
            pl.BlockSpec((BLK, D), lambda i: (i, 0)),
            pl.BlockSpec((BLK, D), lambda i: (i, 0)),
            pl.BlockSpec((BLK, D), lambda i: (i, 0)),
            pl.BlockSpec((8, D), lambda i: (0, 0)),
            pl.BlockSpec((1, 1, BLK), lambda i: (i, 0, 0)),
        ],
        out_specs=[
            pl.BlockSpec((G, D), lambda i: (0, 0)),
            pl.BlockSpec((G, D), lambda i: (0, 0)),
            pl.BlockSpec((G, D), lambda i: (0, 0)),
        ],
        out_shape=[
            jax.ShapeDtypeStruct((G, D), jnp.float32),
            jax.ShapeDtypeStruct((G, D), jnp.float32),
            jax.ShapeDtypeStruct((G, D), jnp.float32),
        ],
    )(p0, p1, p2, awp, batch3)


def _final_body(add_ref, cnt_ref, mx_ref, pw_ref, fc1w_ref, fc1b_ref,
                lng_ref, lnb_ref, fc2w_ref, fc2b_ref, out_ref):
    pw = pw_ref[...]
    m = jnp.max(pw, axis=-1, keepdims=True)
    e = jnp.exp(pw - m)
    sm = e / jnp.sum(e, axis=-1, keepdims=True)
    add = add_ref[...]
    mean = add / jnp.maximum(cnt_ref[...], 1.0)
    pooled = (add * sm[0:1, 0:1] + mean * sm[0:1, 1:2]
              + mx_ref[...] * sm[0:1, 2:3])
    o = jnp.dot(pooled, fc1w_ref[...],
                preferred_element_type=jnp.float32) + fc1b_ref[...]
    o = _gelu(_ln(o, lng_ref[...], lnb_ref[...]))
    o = o + pooled
    out_ref[...] = jnp.dot(o, fc2w_ref[...],
                           preferred_element_type=jnp.float32) + fc2b_ref[...]


def _final(add, cnt, mx, pwp, fc1w, fc1b, lng, lnb, fc2w, fc2b):
    return pl.pallas_call(
        _final_body,
        out_shape=jax.ShapeDtypeStruct((G, LAT), jnp.float32),
    )(add, cnt, mx, pwp, fc1w, fc1b, lng, lnb, fc2w, fc2b)


def _sc_aggregate(x, src3, dst3, zeros):
    mesh = plsc.VectorSubcoreMesh(core_axis_name="c", subcore_axis_name="s")

    @functools.partial(
        pl.kernel,
        out_type=jax.ShapeDtypeStruct((NC, N, D), jnp.float32),
        mesh=mesh,
        scratch_types=[
            pltpu.VMEM((56, EK), jnp.int32),
            pltpu.VMEM((56, EK), jnp.int32),
            pltpu.VMEM((2, EK, D), jnp.float32),
            pltpu.VMEM_SHARED((N, D), jnp.float32),
            pltpu.SemaphoreType.DMA,
            pltpu.SemaphoreType.DMA,
        ],
    )
    def agg(x_hbm, src_hbm, dst_hbm, zero_hbm, out_hbm,
            src_v, dst_v, rows_v, acc_sh, sem0, sem1):
        c = lax.axis_index("c")
        s = lax.axis_index("s")
        w = c * NS + s
        pltpu.sync_copy(zero_hbm, acc_sh.at[pl.ds(s * ZR, ZR)])

        @pl.when(s == 0)
        def _():
            pltpu.sync_copy(zero_hbm.at[pl.ds(0, ZTAIL)],
                            acc_sh.at[pl.ds(NS * ZR, ZTAIL)])

        plsc.subcore_barrier()

        for off, cnt in ((0, 56), (56, 44)):
            pltpu.sync_copy(src_hbm.at[w, pl.ds(off, cnt)],
                            src_v.at[pl.ds(0, cnt)])
            pltpu.sync_copy(dst_hbm.at[w, pl.ds(off, cnt)],
                            dst_v.at[pl.ds(0, cnt)])
            pltpu.async_copy(x_hbm.at[src_v.at[0]], rows_v.at[0], sem0)

            def body(j, carry):
                i0 = 2 * j
                pltpu.async_copy(x_hbm.at[src_v.at[i0 + 1]], rows_v.at[1],
                                 sem1)
                pltpu.make_async_copy(x_hbm.at[src_v.at[i0]],
                                      rows_v.at[0], sem0).wait()
                pltpu.sync_copy(rows_v.at[0], acc_sh.at[dst_v.at[i0]],
                                add=True)

                @pl.when(i0 + 2 < cnt)
                def _():
                    pltpu.async_copy(x_hbm.at[src_v.at[i0 + 2]], rows_v.at[0],
                                     sem0)

                pltpu.make_async_copy(x_hbm.at[src_v.at[i0 + 1]],
                                      rows_v.at[1], sem1).wait()
                pltpu.sync_copy(rows_v.at[1], acc_sh.at[dst_v.at[i0 + 1]],
                                add=True)
                return carry

            lax.fori_loop(0, cnt // 2, body, 0)
        plsc.subcore_barrier()
        pltpu.sync_copy(acc_sh.at[pl.ds(s * ZR, ZR)],
                        out_hbm.at[c, pl.ds(s * ZR, ZR)])

        @pl.when(s == 0)
        def _():
            pltpu.sync_copy(acc_sh.at[pl.ds(NS * ZR, ZTAIL)],
                            out_hbm.at[c, pl.ds(NS * ZR, ZTAIL)])

    return agg(x, src3, dst3, zeros)


def kernel(x, params, edge_index, batch):
    p = params
    src = edge_index[0].reshape(NW, ECH, EK)
    dst = edge_index[1].reshape(NW, ECH, EK)
    batch3 = batch.astype(jnp.int32).reshape(NB, 1, BLK)
    zeros = jnp.zeros((ZR, D), jnp.float32)

    def row(v):
        return v.reshape(1, -1)

    x0, pooled = _pre(x, row(p['in_bn_g']), row(p['in_bn_b']), batch3)
    projs = []
    for l in range(L):
        vnu = _vn_mlp(pooled, p[f'vn_w1_{l}'], row(p[f'vn_b1_{l}']),
                      p[f'vn_w2_{l}'], row(p[f'vn_b2_{l}']))
        xu = _vn_add(x0, vnu, batch3)
        aggr = _sc_aggregate(xu, src, dst, zeros)
        eps_row = jnp.broadcast_to((1.0 + p[f'eps{l}'])[None, None], (1, D))
        x0, proj, pooled = _dense(
            xu, aggr, batch3, eps_row,
            p[f'gate_w{l}'], row(p[f'gate_b{l}']),
            p[f'fc1_w{l}'], row(p[f'fc1_b{l}']),
            row(p[f'n1_g{l}']), row(p[f'n1_b{l}']),
            p[f'fc2_w{l}'], row(p[f'fc2_b{l}']),
            row(p[f'n2_g{l}']), row(p[f'n2_b{l}']),
            row(p[f'bn_g{l}']), row(p[f'bn_b{l}']),
            p[f'jk_w{l}'], row(p[f'jk_b{l}']))
        projs.append(proj)

    awp = jnp.zeros((8, D), jnp.float32).at[:L].set(p['attn_w'])
    pwp = jnp.full((1, D), -jnp.inf, jnp.float32).at[0, :3].set(p['pool_w'])
    add, cnt, mx = _jk_pool(projs[0], projs[1], projs[2], awp, batch3)
    return _final(add, cnt, mx, pwp, p['fc1_w'], row(p['fc1_b']),
                  row(p['ln_g']), row(p['ln_b']), p['fc2_w'],
                  row(p['fc2_b']))


# speedup vs baseline: 6.5979x; 1.0087x over previous
import functools

import jax
import jax.numpy as jnp
import numpy as np
from jax import lax
from jax.experimental import pallas as pl
from jax.experimental.pallas import tpu as pltpu
from jax.experimental.pallas import tpu_sc as plsc

N = 10000
E = 320000
D = 128
L = 3
G = 64
LAT = 64

BLK = 1000
NB = N // BLK

NC = 2
NS = 16
NW = NC * NS
EPW = E // NW
EK = 100
ECH = EPW // EK
ZR = 624
ZTAIL = N - NS * ZR

_INV_BN = 1.0 / np.sqrt(1.0 + 1e-5)


def _gelu(x):
    return x * 0.5 * (1.0 + lax.erf(x * np.float32(1.0 / np.sqrt(2.0))))


def _ln(x, g, b):
    m = jnp.mean(x, axis=-1, keepdims=True)
    v = jnp.mean((x - m) ** 2, axis=-1, keepdims=True)
    return (x - m) / jnp.sqrt(v + 1e-5) * g + b


def _members(batch_ref):
    bv = batch_ref[0]
    gi = lax.broadcasted_iota(jnp.int32, (G, BLK), 0)
    return (gi == bv).astype(jnp.float32)


def _pre_body(x_ref, g_ref, b_ref, batch_ref, x0_ref, pooled_ref):
    i = pl.program_id(0)
    x0 = x_ref[...] * (g_ref[...] * _INV_BN) + b_ref[...]
    x0_ref[...] = x0
    ohT = _members(batch_ref)
    contrib = jnp.dot(ohT, x0, preferred_element_type=jnp.float32)

    @pl.when(i == 0)
    def _():
        pooled_ref[...] = contrib

    @pl.when(i > 0)
    def _():
        pooled_ref[...] += contrib


def _pre(x, g, b, batch3):
    return pl.pallas_call(
        _pre_body,
        grid=(NB,),
        in_specs=[
            pl.BlockSpec((BLK, D), lambda i: (i, 0)),
            pl.BlockSpec((1, D), lambda i: (0, 0)),
            pl.BlockSpec((1, D), lambda i: (0, 0)),
            pl.BlockSpec((1, 1, BLK), lambda i: (i, 0, 0)),
        ],
        out_specs=[
            pl.BlockSpec((BLK, D), lambda i: (i, 0)),
            pl.BlockSpec((G, D), lambda i: (0, 0)),
        ],
        out_shape=[
            jax.ShapeDtypeStruct((N, D), jnp.float32),
            jax.ShapeDtypeStruct((G, D), jnp.float32),
        ],
    )(x, g, b, batch3)


def _vnadd_body(x_ref, pooled_ref, w1_ref, b1_ref, w2_ref, b2_ref, batch_ref,
                out_ref, vnu_s):
    i = pl.program_id(0)

    @pl.when(i == 0)
    def _():
        h = jnp.dot(pooled_ref[...], w1_ref[...],
                    preferred_element_type=jnp.float32) + b1_ref[...]
        h = _gelu(h)
        vnu_s[...] = jnp.dot(h, w2_ref[...],
                             preferred_element_type=jnp.float32) + b2_ref[...]

    ohT = _members(batch_ref)
    upd = lax.dot_general(ohT, vnu_s[...], (((0,), (0,)), ((), ())),
                          preferred_element_type=jnp.float32)
    out_ref[...] = x_ref[...] + upd


def _vn_add(x, pooled, w1, b1, w2, b2, batch3):
    return pl.pallas_call(
        _vnadd_body,
        grid=(NB,),
        in_specs=[
            pl.BlockSpec((BLK, D), lambda i: (i, 0)),
            pl.BlockSpec((G, D), lambda i: (0, 0)),
            pl.BlockSpec((D, D), lambda i: (0, 0)),
            pl.BlockSpec((1, D), lambda i: (0, 0)),
            pl.BlockSpec((D, D), lambda i: (0, 0)),
            pl.BlockSpec((1, D), lambda i: (0, 0)),
            pl.BlockSpec((1, 1, BLK), lambda i: (i, 0, 0)),
        ],
        out_specs=pl.BlockSpec((BLK, D), lambda i: (i, 0)),
        out_shape=jax.ShapeDtypeStruct((N, D), jnp.float32),
        scratch_shapes=[pltpu.VMEM((G, D), jnp.float32)],
    )(x, pooled, w1, b1, w2, b2, batch3)


def _dense_body(x_ref, aggr_ref, batch_ref, eps_ref,
                gw_ref, gb_ref, w1_ref, b1_ref, n1g_ref, n1b_ref,
                w2_ref, b2_ref, n2g_ref, n2b_ref, bng_ref, bnb_ref,
                jkw_ref, jkb_ref,
                xn_ref, proj_ref, pooled_ref):
    i = pl.program_id(0)
    h = x_ref[...] * eps_ref[...] + aggr_ref[0] + aggr_ref[1]
    gate = jax.nn.sigmoid(
        jnp.dot(h, gw_ref[...], preferred_element_type=jnp.float32)
        + gb_ref[...])
    t = jnp.dot(h, w1_ref[...], preferred_element_type=jnp.float32) + b1_ref[...]
    t = _gelu(_ln(t, n1g_ref[...], n1b_ref[...]))
    t = jnp.dot(t, w2_ref[...], preferred_element_type=jnp.float32) + b2_ref[...]
    t = _gelu(_ln(t, n2g_ref[...], n2b_ref[...]))
    xn = gate * t + (1.0 - gate) * h
    xn = xn * (bng_ref[...] * _INV_BN) + bnb_ref[...]
    xn = _gelu(xn)
    xn_ref[...] = xn
    proj_ref[...] = jnp.dot(xn, jkw_ref[...],
                            preferred_element_type=jnp.float32) + jkb_ref[...]
    ohT = _members(batch_ref)
    contrib = jnp.dot(ohT, xn, preferred_element_type=jnp.float32)

    @pl.when(i == 0)
    def _():
        pooled_ref[...] = contrib

    @pl.when(i > 0)
    def _():
        pooled_ref[...] += contrib


def _dense(x, aggr, batch3, eps_row, gw, gb, w1, b1, n1g, n1b,
           w2, b2, n2g, n2b, bng, bnb, jkw, jkb):
    vec = pl.BlockSpec((1, D), lambda i: (0, 0))
    mat = pl.BlockSpec((D, D), lambda i: (0, 0))
    return pl.pallas_call(
        _dense_body,
        grid=(NB,),
        in_specs=[
            pl.BlockSpec((BLK, D), lambda i: (i, 0)),
            pl.BlockSpec((2, BLK, D), lambda i: (0, i, 0)),
            pl.BlockSpec((1, 1, BLK), lambda i: (i, 0, 0)),
            vec, mat, vec, mat, vec, vec, vec, mat, vec, vec, vec, vec, vec,
            mat, vec,
        ],
        out_specs=[
            pl.BlockSpec((BLK, D), lambda i: (i, 0)),
            pl.BlockSpec((BLK, D), lambda i: (i, 0)),
            pl.BlockSpec((G, D), lambda i: (0, 0)),
        ],
        out_shape=[
            jax.ShapeDtypeStruct((N, D), jnp.float32),
            jax.ShapeDtypeStruct((N, D), jnp.float32),
            jax.ShapeDtypeStruct((G, D), jnp.float32),
        ],
    )(x, aggr, batch3, eps_row, gw, gb, w1, b1, n1g, n1b, w2, b2, n2g, n2b,
      bng, bnb, jkw, jkb)


def _jkpool_body(p0_ref, p1_ref, p2_ref, aw_ref, batch_ref, pw_ref,
                 fc1w_ref, fc1b_ref, lng_ref, lnb_ref, fc2w_ref, fc2b_ref,
                 out_ref, add_ref, cnt_ref, mx_ref):
    i = pl.program_id(0)
    a0 = p0_ref[...]
    a1 = p1_ref[...]
    a2 = p2_ref[...]
    aw = aw_ref[...]
    l0 = jnp.sum(a0 * aw[0:1, :], axis=-1, keepdims=True)
    l1 = jnp.sum(a1 * aw[1:2, :], axis=-1, keepdims=True)
    l2 = jnp.sum(a2 * aw[2:3, :], axis=-1, keepdims=True)
    m = jnp.maximum(jnp.maximum(l0, l1), l2)
    e0 = jnp.exp(l0 - m)
    e1 = jnp.exp(l1 - m)
    e2 = jnp.exp(l2 - m)
    s = e0 + e1 + e2
    xf = (e0 * a0 + e1 * a1 + e2 * a2) / s
    ohT = _members(batch_ref)
    addc = jnp.dot(ohT, xf, preferred_element_type=jnp.float32)
    cntc = jnp.dot(ohT, jnp.ones((BLK, D), jnp.float32),
                   preferred_element_type=jnp.float32)

    @pl.when(i == 0)
    def _():
        add_ref[...] = addc
        cnt_ref[...] = cntc
        mx_ref[...] = jnp.full((G, D), -jnp.inf, jnp.float32)

    @pl.when(i > 0)
    def _():
        add_ref[...] += addc
        cnt_ref[...] += cntc

    def body(g, carry):
        eg = (lax.broadcasted_iota(jnp.int32, (G, 1), 0) == g
              ).astype(jnp.float32)
        maskcol = lax.dot_general(ohT, eg, (((0,), (0,)), ((), ())),
                                  preferred_element_type=jnp.float32)
        col = jnp.where(maskcol > 0.5, xf, -jnp.inf)
        gm = jnp.max(col, axis=0, keepdims=True)
        mx_ref[pl.ds(g, 1), :] = jnp.maximum(mx_ref[pl.ds(g, 1), :], gm)
        return carry

    lax.fori_loop(0, G, body, 0)

    @pl.when(i == NB - 1)
    def _():
        pw = pw_ref[...]
        pm = jnp.max(pw, axis=-1, keepdims=True)
        e = jnp.exp(pw - pm)
        sm = e / jnp.sum(e, axis=-1, keepdims=True)
        add = add_ref[...]
        mean = add / jnp.maximum(cnt_ref[...], 1.0)
        pooled = (add * sm[0:1, 0:1] + mean * sm[0:1, 1:2]
                  + mx_ref[...] * sm[0:1, 2:3])
        o = jnp.dot(pooled, fc1w_ref[...],
                    preferred_element_type=jnp.float32) + fc1b_ref[...]
        o = _gelu(_ln(o, lng_ref[...], lnb_ref[...]))
        o = o + pooled
        out_ref[...] = jnp.dot(o, fc2w_ref[...],
                               preferred_element_type=jnp.float32) + fc2b_ref[...]


def _jk_pool_head(p0, p1, p2, awp, batch3, pwp, fc1w, fc1b, lng, lnb,
                  fc2w, fc2b):
    return pl.pallas_call(
        _jkpool_body,
        grid=(NB,),
        in_specs=[
            pl.BlockSpec((BLK, D), lambda i: (i, 0)),
            pl.BlockSpec((BLK, D), lambda i: (i, 0)),
            pl.BlockSpec((BLK, D), lambda i: (i, 0)),
            pl.BlockSpec((8, D), lambda i: (0, 0)),
            pl.BlockSpec((1, 1, BLK), lambda i: (i, 0, 0)),
            pl.BlockSpec((1, D), lambda i: (0, 0)),
            pl.BlockSpec((D, D), lambda i: (0, 0)),
            pl.BlockSpec((1, D), lambda i: (0, 0)),
            pl.BlockSpec((1, D), lambda i: (0, 0)),
            pl.BlockSpec((1, D), lambda i: (0, 0)),
            pl.BlockSpec((D, LAT), lambda i: (0, 0)),
            pl.BlockSpec((1, LAT), lambda i: (0, 0)),
        ],
        out_specs=pl.BlockSpec((G, LAT), lambda i: (0, 0)),
        out_shape=jax.ShapeDtypeStruct((G, LAT), jnp.float32),
        scratch_shapes=[
            pltpu.VMEM((G, D), jnp.float32),
            pltpu.VMEM((G, D), jnp.float32),
            pltpu.VMEM((G, D), jnp.float32),
        ],
    )(p0, p1, p2, awp, batch3, pwp, fc1w, fc1b, lng, lnb, fc2w, fc2b)


def _sc_aggregate(x, src3, dst3, zeros):
    mesh = plsc.VectorSubcoreMesh(core_axis_name="c", subcore_axis_name="s")

    @functools.partial(
        pl.kernel,
        out_type=jax.ShapeDtypeStruct((NC, N, D), jnp.float32),
        mesh=mesh,
        scratch_types=[
            pltpu.VMEM((56, EK), jnp.int32),
            pltpu.VMEM((56, EK), jnp.int32),
            pltpu.VMEM((2, EK, D), jnp.float32),
            pltpu.VMEM_SHARED((N, D), jnp.float32),
            pltpu.SemaphoreType.DMA,
            pltpu.SemaphoreType.DMA,
        ],
    )
    def agg(x_hbm, src_hbm, dst_hbm, zero_hbm, out_hbm,
            src_v, dst_v, rows_v, acc_sh, sem0, sem1):
        c = lax.axis_index("c")
        s = lax.axis_index("s")
        w = c * NS + s
        pltpu.sync_copy(zero_hbm, acc_sh.at[pl.ds(s * ZR, ZR)])

        @pl.when(s == 0)
        def _():
            pltpu.sync_copy(zero_hbm.at[pl.ds(0, ZTAIL)],
                            acc_sh.at[pl.ds(NS * ZR, ZTAIL)])

        plsc.subcore_barrier()

        for off, cnt in ((0, 56), (56, 44)):
            pltpu.sync_copy(src_hbm.at[w, pl.ds(off, cnt)],
                            src_v.at[pl.ds(0, cnt)])
            pltpu.sync_copy(dst_hbm.at[w, pl.ds(off, cnt)],
                            dst_v.at[pl.ds(0, cnt)])
            pltpu.async_copy(x_hbm.at[src_v.at[0]], rows_v.at[0], sem0)

            def body(j, carry):
                i0 = 2 * j
                pltpu.async_copy(x_hbm.at[src_v.at[i0 + 1]], rows_v.at[1],
                                 sem1)
                pltpu.make_async_copy(x_hbm.at[src_v.at[i0]],
                                      rows_v.at[0], sem0).wait()
                pltpu.sync_copy(rows_v.at[0], acc_sh.at[dst_v.at[i0]],
                                add=True)

                @pl.when(i0 + 2 < cnt)
                def _():
                    pltpu.async_copy(x_hbm.at[src_v.at[i0 + 2]], rows_v.at[0],
                                     sem0)

                pltpu.make_async_copy(x_hbm.at[src_v.at[i0 + 1]],
                                      rows_v.at[1], sem1).wait()
                pltpu.sync_copy(rows_v.at[1], acc_sh.at[dst_v.at[i0 + 1]],
                                add=True)
                return carry

            lax.fori_loop(0, cnt // 2, body, 0)
        plsc.subcore_barrier()
        pltpu.sync_copy(acc_sh.at[pl.ds(s * ZR, ZR)],
                        out_hbm.at[c, pl.ds(s * ZR, ZR)])

        @pl.when(s == 0)
        def _():
            pltpu.sync_copy(acc_sh.at[pl.ds(NS * ZR, ZTAIL)],
                            out_hbm.at[c, pl.ds(NS * ZR, ZTAIL)])

    return agg(x, src3, dst3, zeros)


def kernel(x, params, edge_index, batch):
    p = params
    src = edge_index[0].reshape(NW, ECH, EK)
    dst = edge_index[1].reshape(NW, ECH, EK)
    batch3 = batch.astype(jnp.int32).reshape(NB, 1, BLK)
    zeros = jnp.zeros((ZR, D), jnp.float32)

    def row(v):
        return v.reshape(1, -1)

    x0, pooled = _pre(x, row(p['in_bn_g']), row(p['in_bn_b']), batch3)
    projs = []
    for l in range(L):
        xu = _vn_add(x0, pooled, p[f'vn_w1_{l}'], row(p[f'vn_b1_{l}']),
                     p[f'vn_w2_{l}'], row(p[f'vn_b2_{l}']), batch3)
        aggr = _sc_aggregate(xu, src, dst, zeros)
        eps_row = jnp.broadcast_to((1.0 + p[f'eps{l}'])[None, None], (1, D))
        x0, proj, pooled = _dense(
            xu, aggr, batch3, eps_row,
            p[f'gate_w{l}'], row(p[f'gate_b{l}']),
            p[f'fc1_w{l}'], row(p[f'fc1_b{l}']),
            row(p[f'n1_g{l}']), row(p[f'n1_b{l}']),
            p[f'fc2_w{l}'], row(p[f'fc2_b{l}']),
            row(p[f'n2_g{l}']), row(p[f'n2_b{l}']),
            row(p[f'bn_g{l}']), row(p[f'bn_b{l}']),
            p[f'jk_w{l}'], row(p[f'jk_b{l}']))
        projs.append(proj)

    awp = jnp.zeros((8, D), jnp.float32).at[:L].set(p['attn_w'])
    pwp = jnp.full((1, D), -jnp.inf, jnp.float32).at[0, :3].set(p['pool_w'])
    return _jk_pool_head(projs[0], projs[1], projs[2], awp, batch3, pwp,
                         p['fc1_w'], row(p['fc1_b']), row(p['ln_g']),
                         row(p['ln_b']), p['fc2_w'], row(p['fc2_b']))


# speedup vs baseline: 9.7246x; 1.4739x over previous
import functools

import jax
import jax.numpy as jnp
import numpy as np
from jax import lax
from jax.experimental import pallas as pl
from jax.experimental.pallas import tpu as pltpu
from jax.experimental.pallas import tpu_sc as plsc

N = 10000
E = 320000
D = 128
L = 3
G = 64
LAT = 64

BLK = 1000
NB = N // BLK

NC = 2
NS = 16
NW = NC * NS
EPW = E // NW
EK = 100
ECH = EPW // EK
ZR = 624
ZTAIL = N - NS * ZR

_INV_BN = 1.0 / np.sqrt(1.0 + 1e-5)


def _gelu(x):
    return x * 0.5 * (1.0 + lax.erf(x * np.float32(1.0 / np.sqrt(2.0))))


def _ln(x, g, b):
    m = jnp.mean(x, axis=-1, keepdims=True)
    v = jnp.mean((x - m) ** 2, axis=-1, keepdims=True)
    return (x - m) / jnp.sqrt(v + 1e-5) * g + b


def _members(batch_ref):
    bv = batch_ref[0]
    gi = lax.broadcasted_iota(jnp.int32, (G, BLK), 0)
    return (gi == bv).astype(jnp.float32)


def _pre_body(x_ref, g_ref, b_ref, batch_ref, x0_ref, pooled_ref):
    i = pl.program_id(0)
    x0 = x_ref[...] * (g_ref[...] * _INV_BN) + b_ref[...]
    x0_ref[...] = x0
    ohT = _members(batch_ref)
    contrib = jnp.dot(ohT, x0, preferred_element_type=jnp.float32)

    @pl.when(i == 0)
    def _():
        pooled_ref[...] = contrib

    @pl.when(i > 0)
    def _():
        pooled_ref[...] += contrib


def _pre(x, g, b, batch3):
    return pl.pallas_call(
        _pre_body,
        grid=(NB,),
        in_specs=[
            pl.BlockSpec((BLK, D), lambda i: (i, 0)),
            pl.BlockSpec((1, D), lambda i: (0, 0)),
            pl.BlockSpec((1, D), lambda i: (0, 0)),
            pl.BlockSpec((1, 1, BLK), lambda i: (i, 0, 0)),
        ],
        out_specs=[
            pl.BlockSpec((BLK, D), lambda i: (i, 0)),
            pl.BlockSpec((G, D), lambda i: (0, 0)),
        ],
        out_shape=[
            jax.ShapeDtypeStruct((N, D), jnp.float32),
            jax.ShapeDtypeStruct((G, D), jnp.float32),
        ],
    )(x, g, b, batch3)


def _vnadd_body(x_ref, pooled_ref, w1_ref, b1_ref, w2_ref, b2_ref, batch_ref,
                out_ref, vnu_s):
    i = pl.program_id(0)

    @pl.when(i == 0)
    def _():
        h = jnp.dot(pooled_ref[...], w1_ref[...],
                    preferred_element_type=jnp.float32) + b1_ref[...]
        h = _gelu(h)
        vnu_s[...] = jnp.dot(h, w2_ref[...],
                             preferred_element_type=jnp.float32) + b2_ref[...]

    ohT = _members(batch_ref)
    upd = lax.dot_general(ohT, vnu_s[...], (((0,), (0,)), ((), ())),
                          preferred_element_type=jnp.float32)
    out_ref[...] = x_ref[...] + upd


def _vn_add(x, pooled, w1, b1, w2, b2, batch3):
    return pl.pallas_call(
        _vnadd_body,
        grid=(NB,),
        in_specs=[
            pl.BlockSpec((BLK, D), lambda i: (i, 0)),
            pl.BlockSpec((G, D), lambda i: (0, 0)),
            pl.BlockSpec((D, D), lambda i: (0, 0)),
            pl.BlockSpec((1, D), lambda i: (0, 0)),
            pl.BlockSpec((D, D), lambda i: (0, 0)),
            pl.BlockSpec((1, D), lambda i: (0, 0)),
            pl.BlockSpec((1, 1, BLK), lambda i: (i, 0, 0)),
        ],
        out_specs=pl.BlockSpec((BLK, D), lambda i: (i, 0)),
        out_shape=jax.ShapeDtypeStruct((N, D), jnp.float32),
        scratch_shapes=[pltpu.VMEM((G, D), jnp.float32)],
    )(x, pooled, w1, b1, w2, b2, batch3)


def _dense_body(x_ref, aggr_ref, batch_ref, eps_ref,
                gw_ref, gb_ref, w1_ref, b1_ref, n1g_ref, n1b_ref,
                w2_ref, b2_ref, n2g_ref, n2b_ref, bng_ref, bnb_ref,
                jkw_ref, jkb_ref,
                xn_ref, proj_ref, pooled_ref):
    i = pl.program_id(0)
    h = x_ref[...] * eps_ref[...] + aggr_ref[0] + aggr_ref[1]
    gate = jax.nn.sigmoid(
        jnp.dot(h, gw_ref[...], preferred_element_type=jnp.float32)
        + gb_ref[...])
    t = jnp.dot(h, w1_ref[...], preferred_element_type=jnp.float32) + b1_ref[...]
    t = _gelu(_ln(t, n1g_ref[...], n1b_ref[...]))
    t = jnp.dot(t, w2_ref[...], preferred_element_type=jnp.float32) + b2_ref[...]
    t = _gelu(_ln(t, n2g_ref[...], n2b_ref[...]))
    xn = gate * t + (1.0 - gate) * h
    xn = xn * (bng_ref[...] * _INV_BN) + bnb_ref[...]
    xn = _gelu(xn)
    xn_ref[...] = xn
    proj_ref[...] = jnp.dot(xn, jkw_ref[...],
                            preferred_element_type=jnp.float32) + jkb_ref[...]
    ohT = _members(batch_ref)
    contrib = jnp.dot(ohT, xn, preferred_element_type=jnp.float32)

    @pl.when(i == 0)
    def _():
        pooled_ref[...] = contrib

    @pl.when(i > 0)
    def _():
        pooled_ref[...] += contrib


def _dense(x, aggr, batch3, eps_row, gw, gb, w1, b1, n1g, n1b,
           w2, b2, n2g, n2b, bng, bnb, jkw, jkb):
    vec = pl.BlockSpec((1, D), lambda i: (0, 0))
    mat = pl.BlockSpec((D, D), lambda i: (0, 0))
    return pl.pallas_call(
        _dense_body,
        grid=(NB,),
        in_specs=[
            pl.BlockSpec((BLK, D), lambda i: (i, 0)),
            pl.BlockSpec((2, BLK, D), lambda i: (0, i, 0)),
            pl.BlockSpec((1, 1, BLK), lambda i: (i, 0, 0)),
            vec, mat, vec, mat, vec, vec, vec, mat, vec, vec, vec, vec, vec,
            mat, vec,
        ],
        out_specs=[
            pl.BlockSpec((BLK, D), lambda i: (i, 0)),
            pl.BlockSpec((BLK, D), lambda i: (i, 0)),
            pl.BlockSpec((G, D), lambda i: (0, 0)),
        ],
        out_shape=[
            jax.ShapeDtypeStruct((N, D), jnp.float32),
            jax.ShapeDtypeStruct((N, D), jnp.float32),
            jax.ShapeDtypeStruct((G, D), jnp.float32),
        ],
    )(x, aggr, batch3, eps_row, gw, gb, w1, b1, n1g, n1b, w2, b2, n2g, n2b,
      bng, bnb, jkw, jkb)


def _jkpool_body(p0_ref, p1_ref, p2_ref, aw_ref, batch_ref, bsm_ref, pw_ref,
                 fc1w_ref, fc1b_ref, lng_ref, lnb_ref, fc2w_ref, fc2b_ref,
                 out_ref, add_ref, cnt_ref, mx_ref):
    i = pl.program_id(0)
    a0 = p0_ref[...]
    a1 = p1_ref[...]
    a2 = p2_ref[...]
    aw = aw_ref[...]
    l0 = jnp.sum(a0 * aw[0:1, :], axis=-1, keepdims=True)
    l1 = jnp.sum(a1 * aw[1:2, :], axis=-1, keepdims=True)
    l2 = jnp.sum(a2 * aw[2:3, :], axis=-1, keepdims=True)
    m = jnp.maximum(jnp.maximum(l0, l1), l2)
    e0 = jnp.exp(l0 - m)
    e1 = jnp.exp(l1 - m)
    e2 = jnp.exp(l2 - m)
    s = e0 + e1 + e2
    xf = (e0 * a0 + e1 * a1 + e2 * a2) / s
    ohT = _members(batch_ref)
    addc = jnp.dot(ohT, xf, preferred_element_type=jnp.float32)
    cntc = jnp.dot(ohT, jnp.ones((BLK, D), jnp.float32),
                   preferred_element_type=jnp.float32)

    @pl.when(i == 0)
    def _():
        add_ref[...] = addc
        cnt_ref[...] = cntc
        mx_ref[...] = jnp.full((G, D), -jnp.inf, jnp.float32)

    @pl.when(i > 0)
    def _():
        add_ref[...] += addc
        cnt_ref[...] += cntc

    def body(g, carry):
        eg = (lax.broadcasted_iota(jnp.int32, (G, 1), 0) == g
              ).astype(jnp.float32)
        maskcol = lax.dot_general(ohT, eg, (((0,), (0,)), ((), ())),
                                  preferred_element_type=jnp.float32)
        col = jnp.where(maskcol > 0.5, xf, -jnp.inf)
        gm = jnp.max(col, axis=0, keepdims=True)
        mx_ref[pl.ds(g, 1), :] = jnp.maximum(mx_ref[pl.ds(g, 1), :], gm)
        return carry

    g_lo = bsm_ref[0, 0, 0]
    g_hi = bsm_ref[0, 0, BLK - 1]
    lax.fori_loop(g_lo, g_hi + 1, body, 0)

    @pl.when(i == NB - 1)
    def _():
        pw = pw_ref[...]
        pm = jnp.max(pw, axis=-1, keepdims=True)
        e = jnp.exp(pw - pm)
        sm = e / jnp.sum(e, axis=-1, keepdims=True)
        add = add_ref[...]
        mean = add / jnp.maximum(cnt_ref[...], 1.0)
        pooled = (add * sm[0:1, 0:1] + mean * sm[0:1, 1:2]
                  + mx_ref[...] * sm[0:1, 2:3])
        o = jnp.dot(pooled, fc1w_ref[...],
                    preferred_element_type=jnp.float32) + fc1b_ref[...]
        o = _gelu(_ln(o, lng_ref[...], lnb_ref[...]))
        o = o + pooled
        out_ref[...] = jnp.dot(o, fc2w_ref[...],
                               preferred_element_type=jnp.float32) + fc2b_ref[...]


def _jk_pool_head(p0, p1, p2, awp, batch3, pwp, fc1w, fc1b, lng, lnb,
                  fc2w, fc2b):
    return pl.pallas_call(
        _jkpool_body,
        grid=(NB,),
        in_specs=[
            pl.BlockSpec((BLK, D), lambda i: (i, 0)),
            pl.BlockSpec((BLK, D), lambda i: (i, 0)),
            pl.BlockSpec((BLK, D), lambda i: (i, 0)),
            pl.BlockSpec((8, D), lambda i: (0, 0)),
            pl.BlockSpec((1, 1, BLK), lambda i: (i, 0, 0)),
            pl.BlockSpec((1, 1, BLK), lambda i: (i, 0, 0),
                         memory_space=pltpu.SMEM),
            pl.BlockSpec((1, D), lambda i: (0, 0)),
            pl.BlockSpec((D, D), lambda i: (0, 0)),
            pl.BlockSpec((1, D), lambda i: (0, 0)),
            pl.BlockSpec((1, D), lambda i: (0, 0)),
            pl.BlockSpec((1, D), lambda i: (0, 0)),
            pl.BlockSpec((D, LAT), lambda i: (0, 0)),
            pl.BlockSpec((1, LAT), lambda i: (0, 0)),
        ],
        out_specs=pl.BlockSpec((G, LAT), lambda i: (0, 0)),
        out_shape=jax.ShapeDtypeStruct((G, LAT), jnp.float32),
        scratch_shapes=[
            pltpu.VMEM((G, D), jnp.float32),
            pltpu.VMEM((G, D), jnp.float32),
            pltpu.VMEM((G, D), jnp.float32),
        ],
    )(p0, p1, p2, awp, batch3, batch3, pwp, fc1w, fc1b, lng, lnb, fc2w, fc2b)


def _sc_aggregate(x, src3, dst3, zeros):
    mesh = plsc.VectorSubcoreMesh(core_axis_name="c", subcore_axis_name="s")

    @functools.partial(
        pl.kernel,
        out_type=jax.ShapeDtypeStruct((NC, N, D), jnp.float32),
        mesh=mesh,
        scratch_types=[
            pltpu.VMEM((56, EK), jnp.int32),
            pltpu.VMEM((56, EK), jnp.int32),
            pltpu.VMEM((2, EK, D), jnp.float32),
            pltpu.VMEM_SHARED((N, D), jnp.float32),
            pltpu.SemaphoreType.DMA,
            pltpu.SemaphoreType.DMA,
        ],
    )
    def agg(x_hbm, src_hbm, dst_hbm, zero_hbm, out_hbm,
            src_v, dst_v, rows_v, acc_sh, sem0, sem1):
        c = lax.axis_index("c")
        s = lax.axis_index("s")
        w = c * NS + s
        pltpu.sync_copy(zero_hbm, acc_sh.at[pl.ds(s * ZR, ZR)])

        @pl.when(s == 0)
        def _():
            pltpu.sync_copy(zero_hbm.at[pl.ds(0, ZTAIL)],
                            acc_sh.at[pl.ds(NS * ZR, ZTAIL)])

        plsc.subcore_barrier()

        for off, cnt in ((0, 56), (56, 44)):
            pltpu.sync_copy(src_hbm.at[w, pl.ds(off, cnt)],
                            src_v.at[pl.ds(0, cnt)])
            pltpu.sync_copy(dst_hbm.at[w, pl.ds(off, cnt)],
                            dst_v.at[pl.ds(0, cnt)])
            pltpu.async_copy(x_hbm.at[src_v.at[0]], rows_v.at[0], sem0)

            def body(j, carry):
                i0 = 2 * j
                pltpu.async_copy(x_hbm.at[src_v.at[i0 + 1]], rows_v.at[1],
                                 sem1)
                pltpu.make_async_copy(x_hbm.at[src_v.at[i0]],
                                      rows_v.at[0], sem0).wait()
                pltpu.sync_copy(rows_v.at[0], acc_sh.at[dst_v.at[i0]],
                                add=True)

                @pl.when(i0 + 2 < cnt)
                def _():
                    pltpu.async_copy(x_hbm.at[src_v.at[i0 + 2]], rows_v.at[0],
                                     sem0)

                pltpu.make_async_copy(x_hbm.at[src_v.at[i0 + 1]],
                                      rows_v.at[1], sem1).wait()
                pltpu.sync_copy(rows_v.at[1], acc_sh.at[dst_v.at[i0 + 1]],
                                add=True)
                return carry

            lax.fori_loop(0, cnt // 2, body, 0)
        plsc.subcore_barrier()
        pltpu.sync_copy(acc_sh.at[pl.ds(s * ZR, ZR)],
                        out_hbm.at[c, pl.ds(s * ZR, ZR)])

        @pl.when(s == 0)
        def _():
            pltpu.sync_copy(acc_sh.at[pl.ds(NS * ZR, ZTAIL)],
                            out_hbm.at[c, pl.ds(NS * ZR, ZTAIL)])

    return agg(x, src3, dst3, zeros)


def kernel(x, params, edge_index, batch):
    p = params
    src = edge_index[0].reshape(NW, ECH, EK)
    dst = edge_index[1].reshape(NW, ECH, EK)
    batch3 = batch.astype(jnp.int32).reshape(NB, 1, BLK)
    zeros = jnp.zeros((ZR, D), jnp.float32)

    def row(v):
        return v.reshape(1, -1)

    x0, pooled = _pre(x, row(p['in_bn_g']), row(p['in_bn_b']), batch3)
    projs = []
    for l in range(L):
        xu = _vn_add(x0, pooled, p[f'vn_w1_{l}'], row(p[f'vn_b1_{l}']),
                     p[f'vn_w2_{l}'], row(p[f'vn_b2_{l}']), batch3)
        aggr = _sc_aggregate(xu, src, dst, zeros)
        eps_row = jnp.broadcast_to((1.0 + p[f'eps{l}'])[None, None], (1, D))
        x0, proj, pooled = _dense(
            xu, aggr, batch3, eps_row,
            p[f'gate_w{l}'], row(p[f'gate_b{l}']),
            p[f'fc1_w{l}'], row(p[f'fc1_b{l}']),
            row(p[f'n1_g{l}']), row(p[f'n1_b{l}']),
            p[f'fc2_w{l}'], row(p[f'fc2_b{l}']),
            row(p[f'n2_g{l}']), row(p[f'n2_b{l}']),
            row(p[f'bn_g{l}']), row(p[f'bn_b{l}']),
            p[f'jk_w{l}'], row(p[f'jk_b{l}']))
        projs.append(proj)

    awp = jnp.zeros((8, D), jnp.float32).at[:L].set(p['attn_w'])
    pwp = jnp.full((1, D), -jnp.inf, jnp.float32).at[0, :3].set(p['pool_w'])
    return _jk_pool_head(projs[0], projs[1], projs[2], awp, batch3, pwp,
                         p['fc1_w'], row(p['fc1_b']), row(p['ln_g']),
                         row(p['ln_b']), p['fc2_w'], row(p['fc2_b']))
